# Initial kernel scaffold; baseline (speedup 1.0000x reference)
import jax
import jax.numpy as jnp
from jax.experimental import pallas as pl


def kernel(H, D, h_ebd, d_ebd):
    raise NotImplementedError("write your pallas kernel here")


# speedup vs baseline: 1.1116x; 1.1116x over previous
import jax
import jax.numpy as jnp
from jax import lax
from jax.experimental import pallas as pl
from jax.experimental.pallas import tpu as pltpu
from jax.experimental.pallas import tpu_sc as plsc

_N_COMP, _N_NODES, _N_T = 16, 325, 12
_W = _N_COMP * _N_NODES * _N_T
_B = 64
_NC, _NS, _L = 2, 16, 16
_NW = _NC * _NS
_BPW = _B // _NW


def _body(hidx_hbm, didx_hbm, h_hbm, d_hbm, out_hbm,
          hidx_v, didx_v, hrow, drow, sem_h, sem_d):
    wid = lax.axis_index("s") * _NC + lax.axis_index("c")
    for r in range(_BPW):
        b = wid * _BPW + r
        pltpu.sync_copy(hidx_hbm.at[b], hidx_v)
        pltpu.sync_copy(didx_hbm.at[b], didx_v)
        cp_h = pltpu.async_copy(
            h_hbm.at[hidx_v.at[pl.ds(0, 1)]], hrow, sem_h)
        cp_d = pltpu.async_copy(
            d_hbm.at[didx_v.at[pl.ds(0, 1)]], drow, sem_d)
        cp_h.wait()
        cp_d.wait()

        def _add(i, carry):
            sl = pl.ds(i * _L, _L)
            hrow[0, sl] = hrow[0, sl] + drow[0, sl]
            return carry

        lax.fori_loop(0, _W // _L, _add, 0)
        pltpu.sync_copy(hrow, out_hbm.at[pl.ds(b, 1)])


@jax.jit
def _run(H, D, h_ebd, d_ebd):
    hidx = jnp.broadcast_to(H[:, None], (_B, _L)).astype(jnp.int32)
    didx = jnp.broadcast_to(D[:, None], (_B, _L)).astype(jnp.int32)
    mesh = plsc.VectorSubcoreMesh(core_axis_name="c", subcore_axis_name="s")
    out = pl.kernel(
        _body,
        out_type=jax.ShapeDtypeStruct((_B, _W), jnp.float32),
        mesh=mesh,
        compiler_params=pltpu.CompilerParams(use_tc_tiling_on_sc=False),
        scratch_types=[
            pltpu.VMEM((_L,), jnp.int32),
            pltpu.VMEM((_L,), jnp.int32),
            pltpu.VMEM((1, _W), jnp.float32),
            pltpu.VMEM((1, _W), jnp.float32),
            pltpu.SemaphoreType.DMA,
            pltpu.SemaphoreType.DMA,
        ],
    )(hidx, didx, h_ebd, d_ebd)
    return out.reshape(_B, _N_COMP, _N_NODES, _N_T)


def kernel(H, D, h_ebd, d_ebd):
    return _run(H, D, h_ebd, d_ebd)
